# baseline (speedup 1.0000x reference)
import jax
import jax.numpy as jnp
from jax import lax
from jax.experimental import pallas as pl
from jax.experimental.pallas import tpu as pltpu
from jax.experimental.pallas import tpu_sc as plsc

B = 16384
D_IN = 128
D_OUT = D_IN + 6
NC, NS, L = 2, 16, 16
NW = NC * NS
RW = B // NW
CH = 64
NCHUNK = RW // CH
INV_D = 1.0 / D_OUT


def _rsqrt_nr(v):
    i = lax.bitcast_convert_type(v, jnp.int32)
    i = jnp.int32(0x5F3759DF) - lax.shift_right_logical(i, 1)
    y = lax.bitcast_convert_type(i, jnp.float32)
    for _ in range(3):
        y = y * (1.5 - 0.5 * v * y * y)
    return y


def _body(x_hbm, gb_hbm, tab_hbm, out_hbm,
          x_v0, x_v1, o_v0, o_v1, ob_v0, ob_v1, gb_v, tab_v,
          si0, si1, so0, so1):
    x_bufs = (x_v0, x_v1)
    o_bufs = (o_v0, o_v1)
    ob_bufs = (ob_v0, ob_v1)
    si = (si0, si1)
    so = (so0, so1)
    wid = lax.axis_index("s") * NC + lax.axis_index("c")
    h_in0 = pltpu.async_copy(x_hbm.at[pl.ds(wid * RW, CH)], x_v0, si0)
    pltpu.async_copy(tab_hbm, tab_v, si1).wait()
    pltpu.async_copy(gb_hbm, gb_v, si1).wait()
    lanes = lax.iota(jnp.int32, L)
    m_head = lanes < 7
    m6 = lanes < 6
    m_19 = jnp.logical_and(lanes >= 1, lanes <= 9)
    m_ge10 = lanes >= 10
    zeros_i = jnp.zeros((L,), jnp.int32)
    last_i = jnp.full((L,), L - 1, jnp.int32)
    idx6 = jnp.where(m6, lanes + 10, 0)
    idxh = jnp.where(m_head, 0, lanes - 6)
    fz = jnp.zeros((L,), jnp.float32)

    gA = [gb_v[pl.ds(16 * k, L)] for k in range(8)]
    bA = [gb_v[pl.ds(D_OUT + 16 * k, L)] for k in range(8)]
    g6 = gb_v[pl.ds(128, L)]
    b6 = gb_v[pl.ds(D_OUT + 128, L)]

    def compute_chunk(x_v, o_v, ob_v):
        @plsc.parallel_loop(0, CH, unroll=2)
        def rowbody(r):
            xv0 = x_v[r, pl.ds(0, L)]
            la = [x_v[r, pl.ds(16 * k - 6, L)] for k in range(1, 8)]
            ld = x_v[r, pl.ds(112, L)]
            eib = xv0.astype(jnp.int32).at[zeros_i].get(
                mode="promise_in_bounds")
            tix = jnp.where(m_head, (eib + 1) * 7 + lanes, 0)
            ev = plsc.load_gather(tab_v, [tix], mask=m_head)
            ev = jnp.where(m_head, ev, fz)
            x19 = jnp.where(m_19, xv0, fz)
            xtl = jnp.where(m_ge10, ld, fz)
            s8 = ev + x19 + xtl
            q8 = ev * ev + x19 * x19 + xtl * xtl
            for k in range(7):
                s8 = s8 + la[k]
                q8 = q8 + la[k] * la[k]
            tot_s = plsc.cumsum(s8).at[last_i].get(mode="promise_in_bounds")
            tot_q = plsc.cumsum(q8).at[last_i].get(mode="promise_in_bounds")
            mean = tot_s * INV_D
            var = tot_q * INV_D - mean * mean
            rstd = _rsqrt_nr(var + 1e-12)
            sx = xv0.at[idxh].get(mode="promise_in_bounds")
            cat0 = jnp.where(m_head, ev, sx)
            o_v[r, pl.ds(0, L)] = (cat0 - mean) * rstd * gA[0] + bA[0]
            for k in range(7):
                o_v[r, pl.ds(16 * (k + 1), L)] = (la[k] - mean) * rstd * gA[k + 1] + bA[k + 1]
            sh = ld.at[idx6].get(mode="promise_in_bounds")
            ov6 = (sh - mean) * rstd * g6 + b6
            plsc.store_scatter(ob_v, [r + zeros_i, lanes], ov6, mask=m6)

    def row_of(ci):
        return wid * RW + ci * CH

    hin = [None] * NCHUNK
    hout = [None] * NCHUNK
    hin[0] = h_in0
    for ci in range(NCHUNK):
        b = ci % 2
        if ci + 1 < NCHUNK:
            hin[ci + 1] = pltpu.async_copy(
                x_hbm.at[pl.ds(row_of(ci + 1), CH)], x_bufs[1 - b], si[1 - b])
        hin[ci].wait()
        if ci >= 2:
            hout[ci - 2][0].wait()
            hout[ci - 2][1].wait()
        compute_chunk(x_bufs[b], o_bufs[b], ob_bufs[b])
        hout[ci] = (
            pltpu.async_copy(
                o_bufs[b], out_hbm.at[pl.ds(row_of(ci), CH), pl.ds(0, D_IN)],
                so[b]),
            pltpu.async_copy(
                ob_bufs[b], out_hbm.at[pl.ds(row_of(ci), CH), pl.ds(D_IN, 6)],
                so[b]),
        )
    for ci in (NCHUNK - 2, NCHUNK - 1):
        hout[ci][0].wait()
        hout[ci][1].wait()


def kernel(x, table, gamma, beta):
    gb = jnp.concatenate([gamma, beta]).astype(jnp.float32)
    gb = jnp.pad(gb, (0, 20))
    tab = jnp.pad(table.astype(jnp.float32).reshape(-1), (0, 23))
    mesh = plsc.VectorSubcoreMesh(core_axis_name="c", subcore_axis_name="s")
    f = pl.kernel(
        _body,
        out_type=jax.ShapeDtypeStruct((B, D_OUT), jnp.float32),
        mesh=mesh,
        compiler_params=pltpu.CompilerParams(needs_layout_passes=False),
        scratch_types=[
            pltpu.VMEM((CH, D_IN), jnp.float32),
            pltpu.VMEM((CH, D_IN), jnp.float32),
            pltpu.VMEM((CH, D_IN), jnp.float32),
            pltpu.VMEM((CH, D_IN), jnp.float32),
            pltpu.VMEM((CH, 6), jnp.float32),
            pltpu.VMEM((CH, 6), jnp.float32),
            pltpu.VMEM((2 * D_OUT + 20,), jnp.float32),
            pltpu.VMEM((72,), jnp.float32),
            pltpu.SemaphoreType.DMA,
            pltpu.SemaphoreType.DMA,
            pltpu.SemaphoreType.DMA,
            pltpu.SemaphoreType.DMA,
        ],
    )
    return f(x, gb, tab)


# speedup vs baseline: 1.0499x; 1.0499x over previous
import jax
import jax.numpy as jnp
from jax import lax
from jax.experimental import pallas as pl
from jax.experimental.pallas import tpu as pltpu
from jax.experimental.pallas import tpu_sc as plsc

B = 16384
D_IN = 128
D_OUT = D_IN + 6
NC, NS, L = 2, 16, 16
NW = NC * NS
RW = B // NW
CH = 128
NCHUNK = RW // CH
INV_D = 1.0 / D_OUT


def _rsqrt_nr(v):
    i = lax.bitcast_convert_type(v, jnp.int32)
    i = jnp.int32(0x5F3759DF) - lax.shift_right_logical(i, 1)
    y = lax.bitcast_convert_type(i, jnp.float32)
    for _ in range(3):
        y = y * (1.5 - 0.5 * v * y * y)
    return y


def _body(x_hbm, gb_hbm, tab_hbm, out_hbm,
          x_v0, x_v1, o_v0, o_v1, ob_v0, ob_v1, gb_v, tab_v,
          si0, si1, so0, so1):
    x_bufs = (x_v0, x_v1)
    o_bufs = (o_v0, o_v1)
    ob_bufs = (ob_v0, ob_v1)
    si = (si0, si1)
    so = (so0, so1)
    wid = lax.axis_index("s") * NC + lax.axis_index("c")
    h_in0 = pltpu.async_copy(x_hbm.at[pl.ds(wid * RW, CH)], x_v0, si0)
    pltpu.async_copy(tab_hbm, tab_v, si1).wait()
    pltpu.async_copy(gb_hbm, gb_v, si1).wait()
    lanes = lax.iota(jnp.int32, L)
    m_head = lanes < 7
    m6 = lanes < 6
    m_19 = jnp.logical_and(lanes >= 1, lanes <= 9)
    m_ge10 = lanes >= 10
    zeros_i = jnp.zeros((L,), jnp.int32)
    last_i = jnp.full((L,), L - 1, jnp.int32)
    idx6 = jnp.where(m6, lanes + 10, 0)
    idxh = jnp.where(m_head, 0, lanes - 6)
    fz = jnp.zeros((L,), jnp.float32)

    gA = [gb_v[pl.ds(16 * k, L)] for k in range(8)]
    bA = [gb_v[pl.ds(D_OUT + 16 * k, L)] for k in range(8)]
    g6 = gb_v[pl.ds(128, L)]
    b6 = gb_v[pl.ds(D_OUT + 128, L)]

    def compute_chunk(x_v, o_v, ob_v):
        @plsc.parallel_loop(0, CH, unroll=2)
        def rowbody(r):
            xv0 = x_v[r, pl.ds(0, L)]
            la = [x_v[r, pl.ds(16 * k - 6, L)] for k in range(1, 8)]
            ld = x_v[r, pl.ds(112, L)]
            eib = xv0.astype(jnp.int32).at[zeros_i].get(
                mode="promise_in_bounds")
            tix = jnp.where(m_head, (eib + 1) * 7 + lanes, 0)
            ev = plsc.load_gather(tab_v, [tix], mask=m_head)
            ev = jnp.where(m_head, ev, fz)
            x19 = jnp.where(m_19, xv0, fz)
            xtl = jnp.where(m_ge10, ld, fz)
            s8 = ev + x19 + xtl
            q8 = ev * ev + x19 * x19 + xtl * xtl
            for k in range(7):
                s8 = s8 + la[k]
                q8 = q8 + la[k] * la[k]
            tot_s = plsc.cumsum(s8).at[last_i].get(mode="promise_in_bounds")
            tot_q = plsc.cumsum(q8).at[last_i].get(mode="promise_in_bounds")
            mean = tot_s * INV_D
            var = tot_q * INV_D - mean * mean
            rstd = _rsqrt_nr(var + 1e-12)
            sx = xv0.at[idxh].get(mode="promise_in_bounds")
            cat0 = jnp.where(m_head, ev, sx)
            o_v[r, pl.ds(0, L)] = (cat0 - mean) * rstd * gA[0] + bA[0]
            for k in range(7):
                o_v[r, pl.ds(16 * (k + 1), L)] = (la[k] - mean) * rstd * gA[k + 1] + bA[k + 1]
            sh = ld.at[idx6].get(mode="promise_in_bounds")
            ov6 = (sh - mean) * rstd * g6 + b6
            plsc.store_scatter(ob_v, [r + zeros_i, lanes], ov6, mask=m6)

    def row_of(ci):
        return wid * RW + ci * CH

    hin = [None] * NCHUNK
    hout = [None] * NCHUNK
    hin[0] = h_in0
    for ci in range(NCHUNK):
        b = ci % 2
        if ci + 1 < NCHUNK:
            hin[ci + 1] = pltpu.async_copy(
                x_hbm.at[pl.ds(row_of(ci + 1), CH)], x_bufs[1 - b], si[1 - b])
        hin[ci].wait()
        if ci >= 2:
            hout[ci - 2][0].wait()
            hout[ci - 2][1].wait()
        compute_chunk(x_bufs[b], o_bufs[b], ob_bufs[b])
        hout[ci] = (
            pltpu.async_copy(
                o_bufs[b], out_hbm.at[pl.ds(row_of(ci), CH), pl.ds(0, D_IN)],
                so[b]),
            pltpu.async_copy(
                ob_bufs[b], out_hbm.at[pl.ds(row_of(ci), CH), pl.ds(D_IN, 6)],
                so[b]),
        )
    for ci in (NCHUNK - 2, NCHUNK - 1):
        hout[ci][0].wait()
        hout[ci][1].wait()


def kernel(x, table, gamma, beta):
    gb = jnp.concatenate([gamma, beta]).astype(jnp.float32)
    gb = jnp.pad(gb, (0, 20))
    tab = jnp.pad(table.astype(jnp.float32).reshape(-1), (0, 23))
    mesh = plsc.VectorSubcoreMesh(core_axis_name="c", subcore_axis_name="s")
    f = pl.kernel(
        _body,
        out_type=jax.ShapeDtypeStruct((B, D_OUT), jnp.float32),
        mesh=mesh,
        compiler_params=pltpu.CompilerParams(needs_layout_passes=False),
        scratch_types=[
            pltpu.VMEM((CH, D_IN), jnp.float32),
            pltpu.VMEM((CH, D_IN), jnp.float32),
            pltpu.VMEM((CH, D_IN), jnp.float32),
            pltpu.VMEM((CH, D_IN), jnp.float32),
            pltpu.VMEM((CH, 6), jnp.float32),
            pltpu.VMEM((CH, 6), jnp.float32),
            pltpu.VMEM((2 * D_OUT + 20,), jnp.float32),
            pltpu.VMEM((72,), jnp.float32),
            pltpu.SemaphoreType.DMA,
            pltpu.SemaphoreType.DMA,
            pltpu.SemaphoreType.DMA,
            pltpu.SemaphoreType.DMA,
        ],
    )
    return f(x, gb, tab)
